# Initial kernel scaffold; baseline (speedup 1.0000x reference)
import jax
import jax.numpy as jnp
from jax.experimental import pallas as pl


def kernel(x, edge_index, edge_weight, W1, b1, W2, b2, W3, b3, W4, b4):
    raise NotImplementedError("write your pallas kernel here")


# speedup vs baseline: 3.9202x; 3.9202x over previous
import functools

import jax
import jax.numpy as jnp
from jax import lax
from jax.experimental import pallas as pl
from jax.experimental.pallas import tpu as pltpu
from jax.experimental.pallas import tpu_sc as plsc

F32 = jnp.float32
I32 = jnp.int32

NUM_CORES = 2
NUM_SUBCORES = 16
NW = NUM_CORES * NUM_SUBCORES
LANES = 16
BLK = 128
DC = 128


def _splat(vec16, j):
    idx = jnp.full((LANES, 1), j, I32)
    return lax.gather(
        vec16, idx,
        lax.GatherDimensionNumbers(offset_dims=(), collapsed_slice_dims=(0,),
                                   start_index_map=(0,)),
        (1,), mode=lax.GatherScatterMode.PROMISE_IN_BOUNDS)


def _deg_body(npad, nblk, dst_t, ew_t, out0, out1, dst_v, ew_v, zbuf, acc):
    ci = lax.axis_index("c")
    si = lax.axis_index("s")
    w = si * NUM_CORES + ci
    pltpu.sync_copy(dst_t.at[w], dst_v)
    pltpu.sync_copy(ew_t.at[w], ew_v)
    per_tile = npad // NUM_SUBCORES
    for k in range(per_tile // LANES):
        zbuf[pl.ds(k * LANES, LANES)] = jnp.zeros((LANES,), F32)
    pltpu.sync_copy(zbuf, acc.at[pl.ds(si * per_tile, per_tile)])
    plsc.subcore_barrier()

    def body(b, carry):
        pltpu.sync_copy(ew_v.at[b], acc.at[dst_v.at[b]], add=True)
        return carry

    lax.fori_loop(0, nblk, body, 0)
    plsc.subcore_barrier()

    @pl.when(ci == 0)
    def _():
        pltpu.sync_copy(acc.at[pl.ds(si * per_tile, per_tile)],
                        out0.at[pl.ds(si * per_tile, per_tile)])

    @pl.when(ci == 1)
    def _():
        pltpu.sync_copy(acc.at[pl.ds(si * per_tile, per_tile)],
                        out1.at[pl.ds(si * per_tile, per_tile)])


def _make_deg_kernel(npad, nblk):
    mesh = plsc.VectorSubcoreMesh(core_axis_name="c", subcore_axis_name="s")
    return pl.kernel(
        functools.partial(_deg_body, npad, nblk),
        out_type=(jax.ShapeDtypeStruct((npad,), F32),
                  jax.ShapeDtypeStruct((npad,), F32)),
        mesh=mesh,
        scratch_types=[
            pltpu.VMEM((nblk, BLK), I32),
            pltpu.VMEM((nblk, BLK), F32),
            pltpu.VMEM((npad // NUM_SUBCORES,), F32),
            pltpu.VMEM_SHARED((npad,), F32),
        ],
    )


def _agg_body(npr, nblk, nchunks, g3, src_t, dst_t, ew_t, zrows, out,
              src_v, dst_v, ew_v, rows_v, acc, sem):
    ci = lax.axis_index("c")
    si = lax.axis_index("s")
    w = si * NUM_CORES + ci
    pltpu.sync_copy(src_t.at[w], src_v)
    pltpu.sync_copy(dst_t.at[w], dst_v)
    pltpu.sync_copy(ew_t.at[w], ew_v)
    rows_per_tile = npr // NUM_SUBCORES
    row0 = si * rows_per_tile
    for chunk in range(nchunks):
        @pl.when(ci == 0)
        def _():
            pltpu.sync_copy(g3.at[chunk, pl.ds(row0, rows_per_tile)],
                            acc.at[pl.ds(row0, rows_per_tile)])

        @pl.when(ci == 1)
        def _():
            pltpu.sync_copy(zrows, acc.at[pl.ds(row0, rows_per_tile)])

        plsc.subcore_barrier()

        def edge_body(b, carry):
            pltpu.async_copy(g3.at[chunk].at[src_v.at[b]], rows_v, sem).wait()
            for h in range(BLK // LANES):
                ew16 = ew_v[b, pl.ds(h * LANES, LANES)]
                for j in range(LANES):
                    e = h * LANES + j
                    ws = _splat(ew16, j)
                    for f in range(DC // LANES):
                        sl = pl.ds(f * LANES, LANES)
                        rows_v[e, sl] = rows_v[e, sl] * ws
            pltpu.sync_copy(rows_v, acc.at[dst_v.at[b]], add=True)
            return carry

        lax.fori_loop(0, nblk, edge_body, 0)
        plsc.subcore_barrier()
        out_row0 = (ci * nchunks + chunk) * npr + row0
        pltpu.sync_copy(acc.at[pl.ds(row0, rows_per_tile)],
                        out.at[pl.ds(out_row0, rows_per_tile)])


def _make_agg_kernel(npr, nblk, nchunks):
    mesh = plsc.VectorSubcoreMesh(core_axis_name="c", subcore_axis_name="s")
    return pl.kernel(
        functools.partial(_agg_body, npr, nblk, nchunks),
        out_type=jax.ShapeDtypeStruct((NUM_CORES * nchunks * npr, DC), F32),
        mesh=mesh,
        scratch_types=[
            pltpu.VMEM((nblk, BLK), I32),
            pltpu.VMEM((nblk, BLK), I32),
            pltpu.VMEM((nblk, BLK), F32),
            pltpu.VMEM((BLK, DC), F32),
            pltpu.VMEM_SHARED((npr, DC), F32),
            pltpu.SemaphoreType.DMA,
        ],
    )


def _dinv_body(deg0_ref, deg1_ref, out_ref):
    deg = 1.0 + deg0_ref[...] + deg1_ref[...]
    out_ref[...] = lax.rsqrt(deg)


def _scale_x_body(x_ref, dinv_ref, out_ref):
    g = x_ref[...] * dinv_ref[...]
    for c in range(out_ref.shape[0]):
        out_ref[c] = g[:, c * DC:(c + 1) * DC]


def _merge(r_ref):
    nchunks = r_ref.shape[1]
    return jnp.concatenate(
        [r_ref[0, c] + r_ref[1, c] for c in range(nchunks)], axis=-1)


def _layer12_body(r_ref, dinv_ref, w1_ref, b1_ref, w2_ref, out_ref):
    dinv = dinv_ref[...]
    s = _merge(r_ref) * dinv
    h1 = jnp.maximum(jnp.dot(s, w1_ref[...],
                             preferred_element_type=F32) + b1_ref[...], 0.0)
    g2 = jnp.dot(h1, w2_ref[...], preferred_element_type=F32) * dinv
    for c in range(out_ref.shape[0]):
        out_ref[c] = g2[:, c * DC:(c + 1) * DC]


def _layer_mid_body(r_ref, dinv_ref, b_ref, w_ref, out_ref):
    dinv = dinv_ref[...]
    h = jnp.maximum(_merge(r_ref) * dinv + b_ref[...], 0.0)
    g = jnp.dot(h, w_ref[...], preferred_element_type=F32) * dinv
    for c in range(out_ref.shape[0]):
        out_ref[c] = g[:, c * DC:(c + 1) * DC]


def _final_body(r_ref, dinv_ref, b_ref, out_ref):
    out_ref[...] = _merge(r_ref) * dinv_ref[...] + b_ref[...]


def kernel(x, edge_index, edge_weight, W1, b1, W2, b2, W3, b3, W4, b4):
    n, d_in = x.shape
    e = edge_weight.shape[0]
    d_h = W1.shape[1]
    d_mid = W3.shape[1]
    d_out = W4.shape[1]

    per_tile_blk = -(-e // (NW * BLK))
    e_pad = NW * per_tile_blk * BLK
    pad = e_pad - e
    src = jnp.concatenate([edge_index[0].astype(I32), jnp.zeros((pad,), I32)])
    dst = jnp.concatenate([edge_index[1].astype(I32), jnp.zeros((pad,), I32)])
    ew = jnp.concatenate([edge_weight.astype(F32), jnp.zeros((pad,), F32)])
    src_t = src.reshape(NW, per_tile_blk, BLK)
    dst_t = dst.reshape(NW, per_tile_blk, BLK)
    ew_t = ew.reshape(NW, per_tile_blk, BLK)

    npr = -(-n // 128) * 128
    npd = -(-n // (NUM_SUBCORES * LANES)) * (NUM_SUBCORES * LANES)
    zrows = jnp.zeros((npr // NUM_SUBCORES, DC), F32)

    deg0, deg1 = _make_deg_kernel(npd, per_tile_blk)(dst_t, ew_t)
    dinv_full = pl.pallas_call(
        _dinv_body,
        out_shape=jax.ShapeDtypeStruct((npd // 128, 128), F32),
    )(deg0.reshape(npd // 128, 128), deg1.reshape(npd // 128, 128))
    dinv = dinv_full.reshape(npd)[:n].reshape(n, 1)

    ROWS = 400
    grid = (n // ROWS,)
    agg = {}
    for d in {d_in, d_h, d_mid, d_out}:
        agg[d] = _make_agg_kernel(npr, per_tile_blk, d // DC)

    def aggregate(g3, d):
        c = d // DC
        r = agg[d](g3, src_t, dst_t, ew_t, zrows)
        return r.reshape(NUM_CORES, c, npr, DC)

    c_in = d_in // DC
    g0 = pl.pallas_call(
        _scale_x_body,
        out_shape=jax.ShapeDtypeStruct((c_in, npr, DC), F32),
        grid=grid,
        in_specs=[pl.BlockSpec((ROWS, d_in), lambda i: (i, 0)),
                  pl.BlockSpec((ROWS, 1), lambda i: (i, 0))],
        out_specs=pl.BlockSpec((c_in, ROWS, DC), lambda i: (0, i, 0)),
    )(x, dinv)
    r0 = aggregate(g0, d_in)

    c_h = d_h // DC
    g2 = pl.pallas_call(
        _layer12_body,
        out_shape=jax.ShapeDtypeStruct((c_h, npr, DC), F32),
        grid=grid,
        in_specs=[pl.BlockSpec((NUM_CORES, c_in, ROWS, DC),
                               lambda i: (0, 0, i, 0)),
                  pl.BlockSpec((ROWS, 1), lambda i: (i, 0)),
                  pl.BlockSpec((d_in, d_h), lambda i: (0, 0)),
                  pl.BlockSpec((1, d_h), lambda i: (0, 0)),
                  pl.BlockSpec((d_h, d_h), lambda i: (0, 0))],
        out_specs=pl.BlockSpec((c_h, ROWS, DC), lambda i: (0, i, 0)),
    )(r0, dinv, W1, b1.reshape(1, d_h), W2)
    r2 = aggregate(g2, d_h)

    c_mid = d_mid // DC
    g3 = pl.pallas_call(
        _layer_mid_body,
        out_shape=jax.ShapeDtypeStruct((c_mid, npr, DC), F32),
        grid=grid,
        in_specs=[pl.BlockSpec((NUM_CORES, c_h, ROWS, DC),
                               lambda i: (0, 0, i, 0)),
                  pl.BlockSpec((ROWS, 1), lambda i: (i, 0)),
                  pl.BlockSpec((1, d_h), lambda i: (0, 0)),
                  pl.BlockSpec((d_h, d_mid), lambda i: (0, 0))],
        out_specs=pl.BlockSpec((c_mid, ROWS, DC), lambda i: (0, i, 0)),
    )(r2, dinv, b2.reshape(1, d_h), W3)
    r3 = aggregate(g3, d_mid)

    c_out = d_out // DC
    g4 = pl.pallas_call(
        _layer_mid_body,
        out_shape=jax.ShapeDtypeStruct((c_out, npr, DC), F32),
        grid=grid,
        in_specs=[pl.BlockSpec((NUM_CORES, c_mid, ROWS, DC),
                               lambda i: (0, 0, i, 0)),
                  pl.BlockSpec((ROWS, 1), lambda i: (i, 0)),
                  pl.BlockSpec((1, d_mid), lambda i: (0, 0)),
                  pl.BlockSpec((d_mid, d_out), lambda i: (0, 0))],
        out_specs=pl.BlockSpec((c_out, ROWS, DC), lambda i: (0, i, 0)),
    )(r3, dinv, b3.reshape(1, d_mid), W4)
    r4 = aggregate(g4, d_out)

    out = pl.pallas_call(
        _final_body,
        out_shape=jax.ShapeDtypeStruct((n, d_out), F32),
        grid=grid,
        in_specs=[pl.BlockSpec((NUM_CORES, c_out, ROWS, DC),
                               lambda i: (0, 0, i, 0)),
                  pl.BlockSpec((ROWS, 1), lambda i: (i, 0)),
                  pl.BlockSpec((1, d_out), lambda i: (0, 0))],
        out_specs=pl.BlockSpec((ROWS, d_out), lambda i: (i, 0)),
    )(r4, dinv, b4.reshape(1, d_out))
    return out


# speedup vs baseline: 4.4527x; 1.1358x over previous
import functools

import jax
import jax.numpy as jnp
from jax import lax
from jax.experimental import pallas as pl
from jax.experimental.pallas import tpu as pltpu
from jax.experimental.pallas import tpu_sc as plsc

F32 = jnp.float32
I32 = jnp.int32

NUM_CORES = 2
NUM_SUBCORES = 16
NW = NUM_CORES * NUM_SUBCORES
LANES = 16
BLK = 64
DC = 128
ER = 6


def _splat(vec16, j):
    idx = jnp.full((LANES, 1), j, I32)
    return lax.gather(
        vec16, idx,
        lax.GatherDimensionNumbers(offset_dims=(), collapsed_slice_dims=(0,),
                                   start_index_map=(0,)),
        (1,), mode=lax.GatherScatterMode.PROMISE_IN_BOUNDS)


def _deg_body(npad, nblk, dst_t, ew_t, out0, out1, dst_v, ew_v, zbuf, acc):
    ci = lax.axis_index("c")
    si = lax.axis_index("s")
    w = si * NUM_CORES + ci
    pltpu.sync_copy(dst_t.at[w], dst_v)
    pltpu.sync_copy(ew_t.at[w], ew_v)
    per_tile = npad // NUM_SUBCORES
    for k in range(per_tile // LANES):
        zbuf[pl.ds(k * LANES, LANES)] = jnp.zeros((LANES,), F32)
    pltpu.sync_copy(zbuf, acc.at[pl.ds(si * per_tile, per_tile)])
    plsc.subcore_barrier()

    def body(b, carry):
        pltpu.sync_copy(ew_v.at[b], acc.at[dst_v.at[b]], add=True)
        return carry

    lax.fori_loop(0, nblk, body, 0)
    plsc.subcore_barrier()

    @pl.when(ci == 0)
    def _():
        pltpu.sync_copy(acc.at[pl.ds(si * per_tile, per_tile)],
                        out0.at[pl.ds(si * per_tile, per_tile)])

    @pl.when(ci == 1)
    def _():
        pltpu.sync_copy(acc.at[pl.ds(si * per_tile, per_tile)],
                        out1.at[pl.ds(si * per_tile, per_tile)])


def _make_deg_kernel(npad, nblk):
    mesh = plsc.VectorSubcoreMesh(core_axis_name="c", subcore_axis_name="s")
    return pl.kernel(
        functools.partial(_deg_body, npad, nblk),
        out_type=(jax.ShapeDtypeStruct((npad,), F32),
                  jax.ShapeDtypeStruct((npad,), F32)),
        mesh=mesh,
        scratch_types=[
            pltpu.VMEM((nblk, BLK), I32),
            pltpu.VMEM((nblk, BLK), F32),
            pltpu.VMEM((npad // NUM_SUBCORES,), F32),
            pltpu.VMEM_SHARED((npad,), F32),
        ],
    )


def _agg_body(npr, nblk, nchunks, g3, srcs, dsts, ews, zrows, out,
              src_r, dst_r, ew_r, gbuf0, gbuf1, sbuf0, sbuf1, acc,
              gsem0, gsem1, ssem0, ssem1, esem):
    ci = lax.axis_index("c")
    si = lax.axis_index("s")
    w = si * NUM_CORES + ci
    e0 = w * nblk * BLK
    gbuf = (gbuf0, gbuf1)
    sbuf = (sbuf0, sbuf1)
    gsem = (gsem0, gsem1)
    ssem = (ssem0, ssem1)
    rows_per_tile = npr // NUM_SUBCORES
    row0 = si * rows_per_tile

    def eload(b):
        q = lax.rem(b, ER)
        sl = pl.ds(e0 + b * BLK, BLK)
        pltpu.async_copy(srcs.at[sl], src_r.at[q], esem.at[q])
        pltpu.async_copy(dsts.at[sl], dst_r.at[q], esem.at[q])
        pltpu.async_copy(ews.at[sl], ew_r.at[q], esem.at[q])

    def ewait(b):
        q = lax.rem(b, ER)
        sl = pl.ds(e0 + b * BLK, BLK)
        pltpu.make_async_copy(srcs.at[sl], src_r.at[q], esem.at[q]).wait()
        pltpu.make_async_copy(dsts.at[sl], dst_r.at[q], esem.at[q]).wait()
        pltpu.make_async_copy(ews.at[sl], ew_r.at[q], esem.at[q]).wait()

    for b in range(ER):
        eload(b)

    for chunk in range(nchunks):
        @pl.when(ci == 0)
        def _():
            pltpu.sync_copy(g3.at[chunk, pl.ds(row0, rows_per_tile)],
                            acc.at[pl.ds(row0, rows_per_tile)])

        @pl.when(ci == 1)
        def _():
            pltpu.sync_copy(zrows, acc.at[pl.ds(row0, rows_per_tile)])

        plsc.subcore_barrier()

        for b in range(2):
            ewait(b)
            pltpu.async_copy(g3.at[chunk].at[src_r.at[b]], gbuf[b], gsem[b])

        def pair_body(p, carry):
            for k in range(2):
                b = 2 * p + k
                q = lax.rem(b, ER)
                pltpu.make_async_copy(g3.at[chunk].at[src_r.at[q]],
                                      gbuf[k], gsem[k]).wait()

                @pl.when(b >= 2)
                def _():
                    pltpu.make_async_copy(sbuf[k], acc.at[dst_r.at[q]],
                                          ssem[k]).wait()

                @pl.when((b >= 2) & (b + 4 < nblk))
                def _():
                    eload(b + 4)

                for h in range(BLK // LANES):
                    ew16 = ew_r[q, pl.ds(h * LANES, LANES)]
                    for j in range(LANES):
                        e = h * LANES + j
                        ws = _splat(ew16, j)
                        for f in range(DC // LANES):
                            sl = pl.ds(f * LANES, LANES)
                            sbuf[k][e, sl] = gbuf[k][e, sl] * ws

                @pl.when(b + 2 < nblk)
                def _():
                    q2 = lax.rem(b + 2, ER)
                    ewait(b + 2)
                    pltpu.async_copy(g3.at[chunk].at[src_r.at[q2]],
                                     gbuf[k], gsem[k])

                pltpu.async_copy(sbuf[k], acc.at[dst_r.at[q]], ssem[k],
                                 add=True)
            return carry

        lax.fori_loop(0, nblk // 2, pair_body, 0)
        for k in range(2):
            pltpu.make_async_copy(sbuf[k], acc.at[dst_r.at[k]],
                                  ssem[k]).wait()
        plsc.subcore_barrier()
        out_row0 = (ci * nchunks + chunk) * npr + row0
        pltpu.sync_copy(acc.at[pl.ds(row0, rows_per_tile)],
                        out.at[pl.ds(out_row0, rows_per_tile)])
        if chunk + 1 < nchunks:
            for b in range(ER):
                eload(b)


def _make_agg_kernel(npr, nblk, nchunks):
    mesh = plsc.VectorSubcoreMesh(core_axis_name="c", subcore_axis_name="s")
    return pl.kernel(
        functools.partial(_agg_body, npr, nblk, nchunks),
        out_type=jax.ShapeDtypeStruct((NUM_CORES * nchunks * npr, DC), F32),
        mesh=mesh,
        scratch_types=[
            pltpu.VMEM((ER, BLK), I32),
            pltpu.VMEM((ER, BLK), I32),
            pltpu.VMEM((ER, BLK), F32),
            pltpu.VMEM((BLK, DC), F32),
            pltpu.VMEM((BLK, DC), F32),
            pltpu.VMEM((BLK, DC), F32),
            pltpu.VMEM((BLK, DC), F32),
            pltpu.VMEM_SHARED((npr, DC), F32),
            pltpu.SemaphoreType.DMA,
            pltpu.SemaphoreType.DMA,
            pltpu.SemaphoreType.DMA,
            pltpu.SemaphoreType.DMA,
            pltpu.SemaphoreType.DMA((ER,)),
        ],
    )


def _dinv_body(deg0_ref, deg1_ref, out_ref):
    deg = 1.0 + deg0_ref[...] + deg1_ref[...]
    out_ref[...] = lax.rsqrt(deg)


def _scale_x_body(x_ref, dinv_ref, out_ref):
    g = x_ref[...] * dinv_ref[...]
    for c in range(out_ref.shape[0]):
        out_ref[c] = g[:, c * DC:(c + 1) * DC]


def _merge(r_ref):
    nchunks = r_ref.shape[1]
    return jnp.concatenate(
        [r_ref[0, c] + r_ref[1, c] for c in range(nchunks)], axis=-1)


def _layer12_body(r_ref, dinv_ref, w1_ref, b1_ref, w2_ref, out_ref):
    dinv = dinv_ref[...]
    s = _merge(r_ref) * dinv
    h1 = jnp.maximum(jnp.dot(s, w1_ref[...],
                             preferred_element_type=F32) + b1_ref[...], 0.0)
    g2 = jnp.dot(h1, w2_ref[...], preferred_element_type=F32) * dinv
    for c in range(out_ref.shape[0]):
        out_ref[c] = g2[:, c * DC:(c + 1) * DC]


def _layer_mid_body(r_ref, dinv_ref, b_ref, w_ref, out_ref):
    dinv = dinv_ref[...]
    h = jnp.maximum(_merge(r_ref) * dinv + b_ref[...], 0.0)
    g = jnp.dot(h, w_ref[...], preferred_element_type=F32) * dinv
    for c in range(out_ref.shape[0]):
        out_ref[c] = g[:, c * DC:(c + 1) * DC]


def _final_body(r_ref, dinv_ref, b_ref, out_ref):
    out_ref[...] = _merge(r_ref) * dinv_ref[...] + b_ref[...]


def kernel(x, edge_index, edge_weight, W1, b1, W2, b2, W3, b3, W4, b4):
    n, d_in = x.shape
    e = edge_weight.shape[0]
    d_h = W1.shape[1]
    d_mid = W3.shape[1]
    d_out = W4.shape[1]

    per_tile_blk = -(-e // (NW * BLK))
    per_tile_blk += per_tile_blk % 2
    e_pad = NW * per_tile_blk * BLK
    pad = e_pad - e
    src = jnp.concatenate([edge_index[0].astype(I32), jnp.zeros((pad,), I32)])
    dst = jnp.concatenate([edge_index[1].astype(I32), jnp.zeros((pad,), I32)])
    ew = jnp.concatenate([edge_weight.astype(F32), jnp.zeros((pad,), F32)])
    dst_t = dst.reshape(NW, per_tile_blk, BLK)
    ew_t = ew.reshape(NW, per_tile_blk, BLK)

    npr = -(-n // 128) * 128
    npd = -(-n // (NUM_SUBCORES * LANES)) * (NUM_SUBCORES * LANES)
    zrows = jnp.zeros((npr // NUM_SUBCORES, DC), F32)

    deg0, deg1 = _make_deg_kernel(npd, per_tile_blk)(dst_t, ew_t)
    dinv_full = pl.pallas_call(
        _dinv_body,
        out_shape=jax.ShapeDtypeStruct((npd // 128, 128), F32),
    )(deg0.reshape(npd // 128, 128), deg1.reshape(npd // 128, 128))
    dinv = dinv_full.reshape(npd)[:n].reshape(n, 1)

    ROWS = 400
    grid = (n // ROWS,)
    agg = {}
    for d in {d_in, d_h, d_mid, d_out}:
        agg[d] = _make_agg_kernel(npr, per_tile_blk, d // DC)

    def aggregate(g3, d):
        c = d // DC
        r = agg[d](g3, src, dst, ew, zrows)
        return r.reshape(NUM_CORES, c, npr, DC)

    c_in = d_in // DC
    g0 = pl.pallas_call(
        _scale_x_body,
        out_shape=jax.ShapeDtypeStruct((c_in, npr, DC), F32),
        grid=grid,
        in_specs=[pl.BlockSpec((ROWS, d_in), lambda i: (i, 0)),
                  pl.BlockSpec((ROWS, 1), lambda i: (i, 0))],
        out_specs=pl.BlockSpec((c_in, ROWS, DC), lambda i: (0, i, 0)),
    )(x, dinv)
    r0 = aggregate(g0, d_in)

    c_h = d_h // DC
    g2 = pl.pallas_call(
        _layer12_body,
        out_shape=jax.ShapeDtypeStruct((c_h, npr, DC), F32),
        grid=grid,
        in_specs=[pl.BlockSpec((NUM_CORES, c_in, ROWS, DC),
                               lambda i: (0, 0, i, 0)),
                  pl.BlockSpec((ROWS, 1), lambda i: (i, 0)),
                  pl.BlockSpec((d_in, d_h), lambda i: (0, 0)),
                  pl.BlockSpec((1, d_h), lambda i: (0, 0)),
                  pl.BlockSpec((d_h, d_h), lambda i: (0, 0))],
        out_specs=pl.BlockSpec((c_h, ROWS, DC), lambda i: (0, i, 0)),
    )(r0, dinv, W1, b1.reshape(1, d_h), W2)
    r2 = aggregate(g2, d_h)

    c_mid = d_mid // DC
    g3 = pl.pallas_call(
        _layer_mid_body,
        out_shape=jax.ShapeDtypeStruct((c_mid, npr, DC), F32),
        grid=grid,
        in_specs=[pl.BlockSpec((NUM_CORES, c_h, ROWS, DC),
                               lambda i: (0, 0, i, 0)),
                  pl.BlockSpec((ROWS, 1), lambda i: (i, 0)),
                  pl.BlockSpec((1, d_h), lambda i: (0, 0)),
                  pl.BlockSpec((d_h, d_mid), lambda i: (0, 0))],
        out_specs=pl.BlockSpec((c_mid, ROWS, DC), lambda i: (0, i, 0)),
    )(r2, dinv, b2.reshape(1, d_h), W3)
    r3 = aggregate(g3, d_mid)

    c_out = d_out // DC
    g4 = pl.pallas_call(
        _layer_mid_body,
        out_shape=jax.ShapeDtypeStruct((c_out, npr, DC), F32),
        grid=grid,
        in_specs=[pl.BlockSpec((NUM_CORES, c_mid, ROWS, DC),
                               lambda i: (0, 0, i, 0)),
                  pl.BlockSpec((ROWS, 1), lambda i: (i, 0)),
                  pl.BlockSpec((1, d_mid), lambda i: (0, 0)),
                  pl.BlockSpec((d_mid, d_out), lambda i: (0, 0))],
        out_specs=pl.BlockSpec((c_out, ROWS, DC), lambda i: (0, i, 0)),
    )(r3, dinv, b3.reshape(1, d_mid), W4)
    r4 = aggregate(g4, d_out)

    out = pl.pallas_call(
        _final_body,
        out_shape=jax.ShapeDtypeStruct((n, d_out), F32),
        grid=grid,
        in_specs=[pl.BlockSpec((NUM_CORES, c_out, ROWS, DC),
                               lambda i: (0, 0, i, 0)),
                  pl.BlockSpec((ROWS, 1), lambda i: (i, 0)),
                  pl.BlockSpec((1, d_out), lambda i: (0, 0))],
        out_specs=pl.BlockSpec((ROWS, d_out), lambda i: (i, 0)),
    )(r4, dinv, b4.reshape(1, d_out))
    return out
